# Initial kernel scaffold; baseline (speedup 1.0000x reference)
import jax
import jax.numpy as jnp
from jax.experimental import pallas as pl


def kernel(x, table):
    raise NotImplementedError("write your pallas kernel here")


# speedup vs baseline: 1.3002x; 1.3002x over previous
import functools

import jax
import jax.numpy as jnp
from jax import lax
from jax.experimental import pallas as pl
from jax.experimental.pallas import tpu as pltpu
from jax.experimental.pallas import tpu_sc as plsc

B_ROWS, SEQ = 4096, 200
D = 128
B_TOTAL = B_ROWS * SEQ
NC, NS = 2, 16
NW = NC * NS
IDX_W = 128
K = 4
CHUNK = K * IDX_W
X_ROWS = B_TOTAL // IDX_W
ROWS_PER_W = X_ROWS // NW
N_CHUNKS = ROWS_PER_W // K

_mesh = plsc.VectorSubcoreMesh(core_axis_name="c", subcore_axis_name="s")


@functools.partial(
    pl.kernel,
    out_type=jax.ShapeDtypeStruct((B_TOTAL, D), jnp.float32),
    mesh=_mesh,
    scratch_types=[
        pltpu.VMEM((K, IDX_W), jnp.int32),
        pltpu.VMEM((CHUNK, D), jnp.float32),
        pltpu.SemaphoreType.DMA,
    ],
)
def _emb_lookup(x_hbm, table_hbm, out_hbm, idx_v, rows_v, sem):
    wid = lax.axis_index("s") * NC + lax.axis_index("c")
    row_base = wid * ROWS_PER_W

    def body(i, _):
        r0 = row_base + i * K
        pltpu.sync_copy(x_hbm.at[pl.ds(r0, K)], idx_v)
        copies = [
            pltpu.async_copy(
                table_hbm.at[idx_v.at[j]],
                rows_v.at[pl.ds(j * IDX_W, IDX_W)],
                sem,
            )
            for j in range(K)
        ]
        for c in copies:
            c.wait()
        pltpu.sync_copy(rows_v, out_hbm.at[pl.ds(r0 * IDX_W, CHUNK)])
        return 0

    lax.fori_loop(0, N_CHUNKS, body, 0)


def kernel(x, table):
    x2 = x.reshape(X_ROWS, IDX_W).astype(jnp.int32)
    out = _emb_lookup(x2, table)
    return out.reshape(B_ROWS, SEQ, D)
